# Initial kernel scaffold; baseline (speedup 1.0000x reference)
import jax
import jax.numpy as jnp
from jax.experimental import pallas as pl


def kernel(x, emb):
    raise NotImplementedError("write your pallas kernel here")


# speedup vs baseline: 4.2498x; 4.2498x over previous
import functools

import jax
import jax.numpy as jnp
from jax import lax
from jax.experimental import pallas as pl
from jax.experimental.pallas import tpu as pltpu
from jax.experimental.pallas import tpu_sc as plsc

VOCAB = 1000
EMBD = 64
L = 16
NC, NS = 2, 16
NW = NC * NS
W_TOTAL = 1024 * 200
WPW = W_TOTAL // NW
CHUNK = 256
NCHUNK = WPW // CHUNK
GROUPS = CHUNK // L
DBLK = EMBD // L


def _sc_char_embed(x_hbm, emb_hbm, out_hbm, tab_v, idx_v, out_v):
    wid = lax.axis_index("s") * NC + lax.axis_index("c")
    pltpu.sync_copy(emb_hbm, tab_v)

    iota = lax.iota(jnp.int32, L)
    i16 = iota * 16
    i64 = iota * EMBD
    base_w = wid * WPW

    def chunk_body(c, carry):
        w0 = base_w + c * CHUNK
        pltpu.sync_copy(x_hbm.at[pl.ds(w0 * 16, CHUNK * 16)], idx_v)

        def group_body(g, carry_g):
            for db in range(DBLK):
                def j_body(j, accs):
                    xj = plsc.load_gather(idx_v, [i16 + (g * (L * 16) + j)])
                    gidx = xj * EMBD + db * L
                    return tuple(
                        accs[d] + plsc.load_gather(tab_v, [gidx + d])
                        for d in range(L)
                    )

                zero = jnp.zeros((L,), jnp.float32)
                accs = lax.fori_loop(0, 16, j_body, (zero,) * L)
                obase = g * (L * EMBD) + db * L
                for d in range(L):
                    plsc.store_scatter(out_v, [i64 + (obase + d)], accs[d])
            return carry_g

        lax.fori_loop(0, GROUPS, group_body, 0)
        pltpu.sync_copy(out_v, out_hbm.at[pl.ds(w0 * EMBD, CHUNK * EMBD)])
        return carry

    lax.fori_loop(0, NCHUNK, chunk_body, 0)


@jax.jit
def _char_embed_sc(x_flat, emb_flat):
    mesh = plsc.VectorSubcoreMesh(core_axis_name="c", subcore_axis_name="s")
    run = pl.kernel(
        _sc_char_embed,
        out_type=jax.ShapeDtypeStruct((W_TOTAL * EMBD,), jnp.float32),
        mesh=mesh,
        scratch_types=[
            pltpu.VMEM((VOCAB * EMBD,), jnp.float32),
            pltpu.VMEM((CHUNK * 16,), jnp.int32),
            pltpu.VMEM((CHUNK * EMBD,), jnp.float32),
        ],
        compiler_params=pltpu.CompilerParams(needs_layout_passes=False),
    )
    return run(x_flat, emb_flat)


def kernel(x, emb):
    bs, seq, word = x.shape
    out = _char_embed_sc(
        x.reshape(-1).astype(jnp.int32),
        emb.reshape(-1),
    )
    return out.reshape(bs, seq, EMBD)
